# Initial kernel scaffold; baseline (speedup 1.0000x reference)
import jax
import jax.numpy as jnp
from jax.experimental import pallas as pl


def kernel(x, table, ln_weight, ln_bias):
    raise NotImplementedError("write your pallas kernel here")


# speedup vs baseline: 3.4212x; 3.4212x over previous
import functools

import jax
import jax.numpy as jnp
from jax import lax
from jax.experimental import pallas as pl
from jax.experimental.pallas import tpu as pltpu
from jax.experimental.pallas import tpu_sc as plsc

EPS = 1e-5

NC, NS = 2, 16
NW = NC * NS
GRP = 128
KG = 2
BLK = KG * GRP


def _normalize_table(table, w, b):
    V, D = table.shape
    RB = 2000
    assert V % RB == 0

    def body(t_ref, w_ref, b_ref, o_ref):
        e = t_ref[...]
        mu = jnp.mean(e, axis=-1, keepdims=True)
        d = e - mu
        var = jnp.mean(d * d, axis=-1, keepdims=True)
        o_ref[...] = d * lax.rsqrt(var + EPS) * w_ref[...] + b_ref[...]

    return pl.pallas_call(
        body,
        grid=(V // RB,),
        in_specs=[
            pl.BlockSpec((RB, D), lambda i: (i, 0)),
            pl.BlockSpec((1, D), lambda i: (0, 0)),
            pl.BlockSpec((1, D), lambda i: (0, 0)),
        ],
        out_specs=pl.BlockSpec((RB, D), lambda i: (i, 0)),
        out_shape=jax.ShapeDtypeStruct((V, D), jnp.float32),
    )(table, w.reshape(1, D), b.reshape(1, D))


def _gather_mask_sc(ntable, x2d):
    V, D = ntable.shape
    NR, _ = x2d.shape
    N = NR * GRP
    per_w = N // NW
    nblk = per_w // BLK
    assert per_w * NW == N and nblk * BLK == per_w

    mesh = plsc.VectorSubcoreMesh(
        core_axis_name="c", subcore_axis_name="s",
        num_cores=NC, num_subcores=NS)

    @functools.partial(
        pl.kernel,
        out_type=[
            jax.ShapeDtypeStruct((NR, GRP, D), jnp.float32),
            jax.ShapeDtypeStruct((NR, GRP), jnp.int32),
        ],
        mesh=mesh,
        compiler_params=pltpu.CompilerParams(use_tc_tiling_on_sc=False),
        scratch_types=[
            pltpu.VMEM((KG, GRP), jnp.int32),
            pltpu.VMEM((KG, GRP, D), jnp.float32),
            pltpu.VMEM((KG, GRP), jnp.int32),
            pltpu.SemaphoreType.DMA,
        ],
    )
    def k(tab_hbm, x_hbm, out_hbm, mask_hbm, idx_v, rows_v, mask_v, sem):
        wid = lax.axis_index("s") * NC + lax.axis_index("c")
        row0 = wid * (per_w // GRP)

        def block(i, carry):
            row = row0 + i * KG
            pltpu.sync_copy(x_hbm.at[pl.ds(row, KG)], idx_v)
            cps = [
                pltpu.async_copy(tab_hbm.at[idx_v.at[j]], rows_v.at[j], sem)
                for j in range(KG)
            ]
            for j in range(KG):
                for g in range(GRP // 16):
                    iv = idx_v[j, pl.ds(g * 16, 16)]
                    mask_v[j, pl.ds(g * 16, 16)] = jnp.where(
                        iv > 0, jnp.int32(1), jnp.int32(0))
            for cp in cps:
                cp.wait()
            pltpu.sync_copy(mask_v, mask_hbm.at[pl.ds(row, KG)])
            pltpu.sync_copy(rows_v, out_hbm.at[pl.ds(row, KG)])
            return carry

        lax.fori_loop(0, nblk, block, 0)

    return k(ntable, x2d)


def kernel(x, table, ln_weight, ln_bias):
    B, S = x.shape
    V, D = table.shape
    N = B * S
    assert N % (NW * BLK) == 0

    ntable = _normalize_table(table, ln_weight, ln_bias)
    x2d = x.astype(jnp.int32).reshape(N // GRP, GRP)
    out, mask_i32 = _gather_mask_sc(ntable, x2d)
    normed = out.reshape(B, S, D)
    mask = (mask_i32 != 0).reshape(B, S)
    return (normed, mask)
